# baseline (speedup 1.0000x reference)
import jax
import jax.numpy as jnp
from jax.experimental import pallas as pl

_N = 8388608
_SCALE = (0.02, 0.33)

_key = jax.random.key(42)
_k1, _k2 = jax.random.split(_key)
_frac = jax.random.uniform(_k1, (), minval=_SCALE[0], maxval=_SCALE[1])
_ERASE_LEN = int((_N * _frac).astype(jnp.int32))
_BEGIN = int(jax.random.randint(_k2, (), 0, _N - _ERASE_LEN))
_END = _BEGIN + _ERASE_LEN

_COLS = 1024
_ROWS = _N // _COLS
_BR = 256
_GRID = _ROWS // _BR
_BLOCK_ELEMS = _BR * _COLS

_SKIP = [
    (b * _BLOCK_ELEMS >= _BEGIN) and ((b + 1) * _BLOCK_ELEMS <= _END)
    for b in range(_GRID)
]
_SKIP_IDS = [b for b in range(_GRID) if _SKIP[b]]
_S0 = _SKIP_IDS[0] if _SKIP_IDS else -1
_S1 = _SKIP_IDS[-1] if _SKIP_IDS else -2
assert _SKIP_IDS == list(range(_S0, _S1 + 1)) and _S0 > 0


def _body(x_ref, o_ref):
    b = pl.program_id(0)
    base = b * _BLOCK_ELEMS
    row = jax.lax.broadcasted_iota(jnp.int32, (_BR, _COLS), 0)
    col = jax.lax.broadcasted_iota(jnp.int32, (_BR, _COLS), 1)
    flat = base + row * _COLS + col
    mask = (flat >= _BEGIN) & (flat < _END)
    o_ref[...] = jnp.where(mask, jnp.float32(0.0), x_ref[...])


def _in_index_map(b):
    skipped = (b >= _S0) & (b <= _S1)
    return (jnp.where(skipped, _S0 - 1, b), 0)


def kernel(vector):
    x = vector.reshape(_ROWS, _COLS)
    out = pl.pallas_call(
        _body,
        grid=(_GRID,),
        in_specs=[pl.BlockSpec((_BR, _COLS), _in_index_map)],
        out_specs=pl.BlockSpec((_BR, _COLS), lambda b: (b, 0)),
        out_shape=jax.ShapeDtypeStruct((_ROWS, _COLS), jnp.float32),
    )(x)
    return out.reshape(_N)


# speedup vs baseline: 1.1026x; 1.1026x over previous
import jax
import jax.numpy as jnp
from jax.experimental import pallas as pl

_N = 8388608
_SCALE = (0.02, 0.33)

_ERASE_LEN = 1546698
_BEGIN = 3057263
_END = _BEGIN + _ERASE_LEN

_COLS = 1024
_ROWS = _N // _COLS
_BR = 1024
_GRID = _ROWS // _BR
_BLOCK_ELEMS = _BR * _COLS

_SKIP = [
    (b * _BLOCK_ELEMS >= _BEGIN) and ((b + 1) * _BLOCK_ELEMS <= _END)
    for b in range(_GRID)
]
_SKIP_IDS = [b for b in range(_GRID) if _SKIP[b]]
_S0 = _SKIP_IDS[0] if _SKIP_IDS else -1
_S1 = _SKIP_IDS[-1] if _SKIP_IDS else -2
assert _SKIP_IDS == list(range(_S0, _S1 + 1)) and _S0 > 0


def _body(x_ref, o_ref):
    b = pl.program_id(0)
    base = b * _BLOCK_ELEMS
    row = jax.lax.broadcasted_iota(jnp.int32, (_BR, _COLS), 0)
    col = jax.lax.broadcasted_iota(jnp.int32, (_BR, _COLS), 1)
    flat = base + row * _COLS + col
    mask = (flat >= _BEGIN) & (flat < _END)
    o_ref[...] = jnp.where(mask, jnp.float32(0.0), x_ref[...])


def _in_index_map(b):
    skipped = (b >= _S0) & (b <= _S1)
    return (jnp.where(skipped, _S0 - 1, b), 0)


def kernel(vector):
    x = vector.reshape(_ROWS, _COLS)
    out = pl.pallas_call(
        _body,
        grid=(_GRID,),
        in_specs=[pl.BlockSpec((_BR, _COLS), _in_index_map)],
        out_specs=pl.BlockSpec((_BR, _COLS), lambda b: (b, 0)),
        out_shape=jax.ShapeDtypeStruct((_ROWS, _COLS), jnp.float32),
    )(x)
    return out.reshape(_N)


# speedup vs baseline: 2.2758x; 2.0640x over previous
import functools

import jax
import jax.numpy as jnp
from jax import lax
from jax.experimental import pallas as pl
from jax.experimental.pallas import tpu as pltpu
from jax.experimental.pallas import tpu_sc as plsc

_N = 8388608
_ERASE_LEN = 1546698
_BEGIN = 3057263
_END = _BEGIN + _ERASE_LEN

_L = 16
_NC = 2
_NS = 16
_NW = _NC * _NS
_CH = 32768
_NCHUNK = _N // _CH
_CPW = _NCHUNK // _NW
_ZB = 8192

_MB = _BEGIN // _CH
_ME = _END // _CH
_BA = (_BEGIN + _L - 1) // _L * _L
_EA = _END // _L * _L
_B_IN = _BEGIN - _MB * _CH
_BA_IN = _BA - _MB * _CH
_E_IN = _END - _ME * _CH
_EA_IN = _EA - _ME * _CH
_FE_LO = _MB + 1
_FE_HI = _ME

assert _MB < _ME and _BA_IN % _L == 0 and _EA_IN % _L == 0
assert _BA_IN < _CH and _EA_IN + _L <= _CH


def _zero_fill(o_hbm, zbuf, start, total):
    off = 0
    while off < total:
        n = min(_ZB, total - off)
        pltpu.sync_copy(zbuf.at[pl.ds(0, n)], o_hbm.at[pl.ds(start + off, n)])
        off += n


def _sc_body(x_hbm, o_hbm, buf, zbuf):
    cid = lax.axis_index("c")
    sid = lax.axis_index("s")
    wid = sid * _NC + cid

    def _zb(i, carry):
        zbuf[pl.ds(i * _L, _L)] = jnp.zeros((_L,), jnp.float32)
        return carry

    lax.fori_loop(0, _ZB // _L, _zb, 0)
    lane = lax.broadcasted_iota(jnp.int32, (_L,), 0)

    for j in range(_CPW):
        m = wid + _NW * j
        lo = m * _CH
        fully_erased = (m >= _FE_LO) & (m < _FE_HI)
        is_b = m == _MB
        is_e = m == _ME
        plain = jnp.logical_not(fully_erased | is_b | is_e)

        @pl.when(jnp.logical_not(fully_erased))
        def _load():
            pltpu.sync_copy(x_hbm.at[pl.ds(lo, _CH)], buf)

        @pl.when(plain)
        def _store_plain():
            pltpu.sync_copy(buf, o_hbm.at[pl.ds(lo, _CH)])

        @pl.when(fully_erased)
        def _store_zeros():
            _zero_fill(o_hbm, zbuf, lo, _CH)

        @pl.when(is_b)
        def _store_begin():
            base = _B_IN // _L * _L
            v = buf[pl.ds(base, _L)]
            buf[pl.ds(base, _L)] = jnp.where(
                lane >= _B_IN - base, jnp.float32(0.0), v)
            pltpu.sync_copy(buf.at[pl.ds(0, _BA_IN)],
                            o_hbm.at[pl.ds(lo, _BA_IN)])
            _zero_fill(o_hbm, zbuf, lo + _BA_IN, _CH - _BA_IN)

        @pl.when(is_e)
        def _store_end():
            v = buf[pl.ds(_EA_IN, _L)]
            buf[pl.ds(_EA_IN, _L)] = jnp.where(
                lane < _E_IN - _EA_IN, jnp.float32(0.0), v)
            _zero_fill(o_hbm, zbuf, lo, _EA_IN)
            pltpu.sync_copy(buf.at[pl.ds(_EA_IN, _CH - _EA_IN)],
                            o_hbm.at[pl.ds(lo + _EA_IN, _CH - _EA_IN)])


_sc_call = functools.partial(
    pl.kernel,
    out_type=jax.ShapeDtypeStruct((_N,), jnp.float32),
    mesh=plsc.VectorSubcoreMesh(core_axis_name="c", subcore_axis_name="s"),
    scratch_types=[
        pltpu.VMEM((_CH,), jnp.float32),
        pltpu.VMEM((_ZB,), jnp.float32),
    ],
)(_sc_body)


def kernel(vector):
    return _sc_call(vector)


# speedup vs baseline: 2.4923x; 1.0951x over previous
import functools

import jax
import jax.numpy as jnp
from jax import lax
from jax.experimental import pallas as pl
from jax.experimental.pallas import tpu as pltpu
from jax.experimental.pallas import tpu_sc as plsc

_N = 8388608
_ERASE_LEN = 1546698
_BEGIN = 3057263
_END = _BEGIN + _ERASE_LEN

_L = 16
_NC = 2
_NS = 16
_NW = _NC * _NS
_CH = 32768
_NCHUNK = _N // _CH
_CPW = _NCHUNK // _NW
_ZB = 8192

_MB = _BEGIN // _CH
_ME = _END // _CH
_BA = (_BEGIN + _L - 1) // _L * _L
_EA = _END // _L * _L
_B_IN = _BEGIN - _MB * _CH
_BA_IN = _BA - _MB * _CH
_E_IN = _END - _ME * _CH
_EA_IN = _EA - _ME * _CH
_FE_LO = _MB + 1
_FE_HI = _ME

assert _MB < _ME and _BA_IN % _L == 0 and _EA_IN % _L == 0
assert _BA_IN < _CH and _EA_IN + _L <= _CH


def _zero_fill(o_hbm, zbuf, sem, start, total):
    off = 0
    while off < total:
        n = min(_ZB, total - off)
        pltpu.make_async_copy(
            zbuf.at[pl.ds(0, n)], o_hbm.at[pl.ds(start + off, n)], sem
        ).start()
        off += n


def _sc_body(x_hbm, o_hbm, buf0, buf1, zbuf, in_sems, out_sems):
    cid = lax.axis_index("c")
    sid = lax.axis_index("s")
    wid = sid * _NC + cid
    bufs = (buf0, buf1)

    def _zb(i, carry):
        z = jnp.zeros((_L,), jnp.float32)
        base = i * (4 * _L)
        zbuf[pl.ds(base, _L)] = z
        zbuf[pl.ds(base + _L, _L)] = z
        zbuf[pl.ds(base + 2 * _L, _L)] = z
        zbuf[pl.ds(base + 3 * _L, _L)] = z
        return carry

    lax.fori_loop(0, _ZB // (4 * _L), _zb, 0)
    lane = lax.broadcasted_iota(jnp.int32, (_L,), 0)

    def chunk_idx(j):
        return wid + _NW * j

    def not_full(j):
        m = chunk_idx(j)
        return jnp.logical_not((m >= _FE_LO) & (m < _FE_HI))

    def in_desc(j):
        m = chunk_idx(j)
        return pltpu.make_async_copy(
            x_hbm.at[pl.ds(m * _CH, _CH)], bufs[j % 2], in_sems.at[j % 2]
        )

    def drain_out(par):
        pltpu.make_async_copy(
            x_hbm.at[pl.ds(0, _CH)], bufs[par], out_sems.at[par]
        ).wait()

    @pl.when(not_full(0))
    def _():
        in_desc(0).start()

    for j in range(_CPW):
        par = j % 2
        buf = bufs[par]
        osem = out_sems.at[par]
        m = chunk_idx(j)
        lo = m * _CH
        fully_erased = (m >= _FE_LO) & (m < _FE_HI)
        is_b = m == _MB
        is_e = m == _ME
        plain = jnp.logical_not(fully_erased | is_b | is_e)

        @pl.when(jnp.logical_not(fully_erased))
        def _wait_in():
            in_desc(j).wait()

        @pl.when(plain)
        def _store_plain():
            pltpu.make_async_copy(buf, o_hbm.at[pl.ds(lo, _CH)], osem).start()

        @pl.when(fully_erased)
        def _store_zeros():
            _zero_fill(o_hbm, zbuf, osem, lo, _CH)

        @pl.when(is_b)
        def _store_begin():
            base = _B_IN // _L * _L
            v = buf[pl.ds(base, _L)]
            buf[pl.ds(base, _L)] = jnp.where(
                lane >= _B_IN - base, jnp.float32(0.0), v)
            pltpu.make_async_copy(
                buf.at[pl.ds(0, _BA_IN)], o_hbm.at[pl.ds(lo, _BA_IN)], osem
            ).start()
            _zero_fill(o_hbm, zbuf, osem, lo + _BA_IN, _CH - _BA_IN)

        @pl.when(is_e)
        def _store_end():
            v = buf[pl.ds(_EA_IN, _L)]
            buf[pl.ds(_EA_IN, _L)] = jnp.where(
                lane < _E_IN - _EA_IN, jnp.float32(0.0), v)
            _zero_fill(o_hbm, zbuf, osem, lo, _EA_IN)
            pltpu.make_async_copy(
                buf.at[pl.ds(_EA_IN, _CH - _EA_IN)],
                o_hbm.at[pl.ds(lo + _EA_IN, _CH - _EA_IN)], osem,
            ).start()

        if j + 1 < _CPW:
            if j >= 1:
                drain_out(1 - par)

            @pl.when(not_full(j + 1))
            def _start_next_in():
                in_desc(j + 1).start()

    drain_out(0)
    drain_out(1)


_sc_call = functools.partial(
    pl.kernel,
    out_type=jax.ShapeDtypeStruct((_N,), jnp.float32),
    mesh=plsc.VectorSubcoreMesh(core_axis_name="c", subcore_axis_name="s"),
    scratch_types=[
        pltpu.VMEM((_CH,), jnp.float32),
        pltpu.VMEM((_CH,), jnp.float32),
        pltpu.VMEM((_ZB,), jnp.float32),
        pltpu.SemaphoreType.DMA((2,)),
        pltpu.SemaphoreType.DMA((2,)),
    ],
)(_sc_body)


def kernel(vector):
    return _sc_call(vector)


# speedup vs baseline: 2.6146x; 1.0491x over previous
import functools

import jax
import jax.numpy as jnp
from jax import lax
from jax.experimental import pallas as pl
from jax.experimental.pallas import tpu as pltpu
from jax.experimental.pallas import tpu_sc as plsc

_N = 8388608
_ERASE_LEN = 1546698
_BEGIN = 3057263
_END = _BEGIN + _ERASE_LEN

_L = 16
_NC = 2
_NS = 16
_NW = _NC * _NS
_CH = 16384
_NBUF = 4
_NCHUNK = _N // _CH
_CPW = _NCHUNK // _NW
_ZB = 8192

_MB = _BEGIN // _CH
_ME = _END // _CH
_BA = (_BEGIN + _L - 1) // _L * _L
_EA = _END // _L * _L
_B_IN = _BEGIN - _MB * _CH
_BA_IN = _BA - _MB * _CH
_E_IN = _END - _ME * _CH
_EA_IN = _EA - _ME * _CH
_FE_LO = _MB + 1
_FE_HI = _ME

assert _N % _CH == 0 and _NCHUNK % _NW == 0 and _CH % _ZB == 0
assert _MB < _ME and _BA_IN % _L == 0 and _EA_IN % _L == 0
assert 0 < _BA_IN < _CH and _EA_IN + _L <= _CH


def _zero_fill(o_hbm, zbuf, sem, start, total):
    off = 0
    while off < total:
        n = min(_ZB, total - off)
        pltpu.make_async_copy(
            zbuf.at[pl.ds(0, n)], o_hbm.at[pl.ds(start + off, n)], sem
        ).start()
        off += n


def _sc_body(x_hbm, o_hbm, buf0, buf1, buf2, buf3, zbuf, in_sems, out_sems):
    cid = lax.axis_index("c")
    sid = lax.axis_index("s")
    wid = sid * _NC + cid
    bufs = (buf0, buf1, buf2, buf3)

    def _zb(i, carry):
        z = jnp.zeros((_L,), jnp.float32)
        base = i * (4 * _L)
        zbuf[pl.ds(base, _L)] = z
        zbuf[pl.ds(base + _L, _L)] = z
        zbuf[pl.ds(base + 2 * _L, _L)] = z
        zbuf[pl.ds(base + 3 * _L, _L)] = z
        return carry

    lax.fori_loop(0, _ZB // (4 * _L), _zb, 0)
    lane = lax.broadcasted_iota(jnp.int32, (_L,), 0)

    def chunk_idx(j):
        return wid + _NW * j

    def not_full(j):
        m = chunk_idx(j)
        return jnp.logical_not((m >= _FE_LO) & (m < _FE_HI))

    def in_desc(j):
        b = j % _NBUF
        return pltpu.make_async_copy(
            x_hbm.at[pl.ds(chunk_idx(j) * _CH, _CH)], bufs[b],
            in_sems.at[b])

    def start_in(j):
        @pl.when(not_full(j))
        def _():
            in_desc(j).start()

    def drain_out(b):
        pltpu.make_async_copy(
            x_hbm.at[pl.ds(0, _CH)], bufs[b], out_sems.at[b]
        ).wait()

    for k in range(min(_NBUF - 1, _CPW)):
        start_in(k)

    for j in range(_CPW):
        b = j % _NBUF
        buf = bufs[b]
        osem = out_sems.at[b]
        m = chunk_idx(j)
        lo = m * _CH
        fully_erased = (m >= _FE_LO) & (m < _FE_HI)
        is_b = m == _MB
        is_e = m == _ME
        plain = jnp.logical_not(fully_erased | is_b | is_e)

        @pl.when(jnp.logical_not(fully_erased))
        def _wait_in():
            in_desc(j).wait()

        @pl.when(plain)
        def _store_plain():
            pltpu.make_async_copy(buf, o_hbm.at[pl.ds(lo, _CH)], osem).start()

        @pl.when(fully_erased)
        def _store_zeros():
            _zero_fill(o_hbm, zbuf, osem, lo, _CH)

        @pl.when(is_b)
        def _store_begin():
            base = _B_IN // _L * _L
            v = buf[pl.ds(base, _L)]
            buf[pl.ds(base, _L)] = jnp.where(
                lane >= _B_IN - base, jnp.float32(0.0), v)
            pltpu.make_async_copy(
                buf.at[pl.ds(0, _BA_IN)], o_hbm.at[pl.ds(lo, _BA_IN)], osem
            ).start()
            _zero_fill(o_hbm, zbuf, osem, lo + _BA_IN, _CH - _BA_IN)

        @pl.when(is_e)
        def _store_end():
            v = buf[pl.ds(_EA_IN, _L)]
            buf[pl.ds(_EA_IN, _L)] = jnp.where(
                lane < _E_IN - _EA_IN, jnp.float32(0.0), v)
            _zero_fill(o_hbm, zbuf, osem, lo, _EA_IN)
            pltpu.make_async_copy(
                buf.at[pl.ds(_EA_IN, _CH - _EA_IN)],
                o_hbm.at[pl.ds(lo + _EA_IN, _CH - _EA_IN)], osem,
            ).start()

        nxt = j + _NBUF - 1
        if nxt < _CPW:
            if j >= 1:
                drain_out((j - 1) % _NBUF)
            start_in(nxt)

    for j in range(max(_CPW - _NBUF, 0), _CPW):
        drain_out(j % _NBUF)


_sc_call = functools.partial(
    pl.kernel,
    out_type=jax.ShapeDtypeStruct((_N,), jnp.float32),
    mesh=plsc.VectorSubcoreMesh(core_axis_name="c", subcore_axis_name="s"),
    scratch_types=[
        pltpu.VMEM((_CH,), jnp.float32),
        pltpu.VMEM((_CH,), jnp.float32),
        pltpu.VMEM((_CH,), jnp.float32),
        pltpu.VMEM((_CH,), jnp.float32),
        pltpu.VMEM((_ZB,), jnp.float32),
        pltpu.SemaphoreType.DMA((_NBUF,)),
        pltpu.SemaphoreType.DMA((_NBUF,)),
    ],
)(_sc_body)


def kernel(vector):
    return _sc_call(vector)


# speedup vs baseline: 2.6787x; 1.0245x over previous
import functools

import jax
import jax.numpy as jnp
from jax import lax
from jax.experimental import pallas as pl
from jax.experimental.pallas import tpu as pltpu
from jax.experimental.pallas import tpu_sc as plsc

_N = 8388608
_ERASE_LEN = 1546698
_BEGIN = 3057263
_END = _BEGIN + _ERASE_LEN

_L = 16
_NC = 2
_NS = 16
_NW = _NC * _NS
_CH = 16384
_NBUF = 7
_NCHUNK = _N // _CH
_CPW = _NCHUNK // _NW
_ZB = 8192

_MB = _BEGIN // _CH
_ME = _END // _CH
_BA = (_BEGIN + _L - 1) // _L * _L
_EA = _END // _L * _L
_B_IN = _BEGIN - _MB * _CH
_BA_IN = _BA - _MB * _CH
_E_IN = _END - _ME * _CH
_EA_IN = _EA - _ME * _CH
_FE_LO = _MB + 1
_FE_HI = _ME

assert _N % _CH == 0 and _NCHUNK % _NW == 0 and _CH % _ZB == 0
assert _MB < _ME and _BA_IN % _L == 0 and _EA_IN % _L == 0
assert 0 < _BA_IN < _CH and _EA_IN + _L <= _CH


def _zero_fill(o_hbm, zbuf, sem, start, total):
    off = 0
    while off < total:
        n = min(_ZB, total - off)
        pltpu.make_async_copy(
            zbuf.at[pl.ds(0, n)], o_hbm.at[pl.ds(start + off, n)], sem
        ).start()
        off += n


def _sc_body(x_hbm, o_hbm, buf0, buf1, buf2, buf3, buf4, buf5, buf6, zbuf,
             in_sems, out_sems):
    cid = lax.axis_index("c")
    sid = lax.axis_index("s")
    wid = sid * _NC + cid
    bufs = (buf0, buf1, buf2, buf3, buf4, buf5, buf6)

    def _zb(i, carry):
        z = jnp.zeros((_L,), jnp.float32)
        base = i * (4 * _L)
        zbuf[pl.ds(base, _L)] = z
        zbuf[pl.ds(base + _L, _L)] = z
        zbuf[pl.ds(base + 2 * _L, _L)] = z
        zbuf[pl.ds(base + 3 * _L, _L)] = z
        return carry

    lax.fori_loop(0, _ZB // (4 * _L), _zb, 0)
    lane = lax.broadcasted_iota(jnp.int32, (_L,), 0)

    def chunk_idx(j):
        return wid + _NW * j

    def not_full(j):
        m = chunk_idx(j)
        return jnp.logical_not((m >= _FE_LO) & (m < _FE_HI))

    def in_desc(j):
        b = j % _NBUF
        return pltpu.make_async_copy(
            x_hbm.at[pl.ds(chunk_idx(j) * _CH, _CH)], bufs[b],
            in_sems.at[b])

    def start_in(j):
        @pl.when(not_full(j))
        def _():
            in_desc(j).start()

    def drain_out(b):
        pltpu.make_async_copy(
            x_hbm.at[pl.ds(0, _CH)], bufs[b], out_sems.at[b]
        ).wait()

    for k in range(min(_NBUF - 1, _CPW)):
        start_in(k)

    for j in range(_CPW):
        b = j % _NBUF
        buf = bufs[b]
        osem = out_sems.at[b]
        m = chunk_idx(j)
        lo = m * _CH
        fully_erased = (m >= _FE_LO) & (m < _FE_HI)
        is_b = m == _MB
        is_e = m == _ME
        plain = jnp.logical_not(fully_erased | is_b | is_e)

        @pl.when(jnp.logical_not(fully_erased))
        def _wait_in():
            in_desc(j).wait()

        @pl.when(plain)
        def _store_plain():
            pltpu.make_async_copy(buf, o_hbm.at[pl.ds(lo, _CH)], osem).start()

        @pl.when(fully_erased)
        def _store_zeros():
            _zero_fill(o_hbm, zbuf, osem, lo, _CH)

        @pl.when(is_b)
        def _store_begin():
            base = _B_IN // _L * _L
            v = buf[pl.ds(base, _L)]
            buf[pl.ds(base, _L)] = jnp.where(
                lane >= _B_IN - base, jnp.float32(0.0), v)
            pltpu.make_async_copy(
                buf.at[pl.ds(0, _BA_IN)], o_hbm.at[pl.ds(lo, _BA_IN)], osem
            ).start()
            _zero_fill(o_hbm, zbuf, osem, lo + _BA_IN, _CH - _BA_IN)

        @pl.when(is_e)
        def _store_end():
            v = buf[pl.ds(_EA_IN, _L)]
            buf[pl.ds(_EA_IN, _L)] = jnp.where(
                lane < _E_IN - _EA_IN, jnp.float32(0.0), v)
            _zero_fill(o_hbm, zbuf, osem, lo, _EA_IN)
            pltpu.make_async_copy(
                buf.at[pl.ds(_EA_IN, _CH - _EA_IN)],
                o_hbm.at[pl.ds(lo + _EA_IN, _CH - _EA_IN)], osem,
            ).start()

        nxt = j + _NBUF - 1
        if nxt < _CPW:
            if j >= 1:
                drain_out((j - 1) % _NBUF)
            start_in(nxt)

    for j in range(max(_CPW - _NBUF, 0), _CPW):
        drain_out(j % _NBUF)


_sc_call = functools.partial(
    pl.kernel,
    out_type=jax.ShapeDtypeStruct((_N,), jnp.float32),
    mesh=plsc.VectorSubcoreMesh(core_axis_name="c", subcore_axis_name="s"),
    scratch_types=[
        pltpu.VMEM((_CH,), jnp.float32),
        pltpu.VMEM((_CH,), jnp.float32),
        pltpu.VMEM((_CH,), jnp.float32),
        pltpu.VMEM((_CH,), jnp.float32),
        pltpu.VMEM((_CH,), jnp.float32),
        pltpu.VMEM((_CH,), jnp.float32),
        pltpu.VMEM((_CH,), jnp.float32),
        pltpu.VMEM((_ZB,), jnp.float32),
        pltpu.SemaphoreType.DMA((_NBUF,)),
        pltpu.SemaphoreType.DMA((_NBUF,)),
    ],
)(_sc_body)


def kernel(vector):
    return _sc_call(vector)


# speedup vs baseline: 2.6920x; 1.0049x over previous
import functools

import jax
import jax.numpy as jnp
from jax import lax
from jax.experimental import pallas as pl
from jax.experimental.pallas import tpu as pltpu
from jax.experimental.pallas import tpu_sc as plsc

_N = 8388608
_ERASE_LEN = 1546698
_BEGIN = 3057263
_END = _BEGIN + _ERASE_LEN

_L = 16
_NC = 2
_NS = 16
_NW = _NC * _NS
_CH = 8192
_NBUF = 14
_NCHUNK = _N // _CH
_CPW = _NCHUNK // _NW
_ZB = 8192

_MB = _BEGIN // _CH
_ME = _END // _CH
_BA = (_BEGIN + _L - 1) // _L * _L
_EA = _END // _L * _L
_B_IN = _BEGIN - _MB * _CH
_BA_IN = _BA - _MB * _CH
_E_IN = _END - _ME * _CH
_EA_IN = _EA - _ME * _CH
_FE_LO = _MB + 1
_FE_HI = _ME

assert _N % _CH == 0 and _NCHUNK % _NW == 0 and _CH % _ZB == 0
assert _MB < _ME and _BA_IN % _L == 0 and _EA_IN % _L == 0
assert 0 < _BA_IN < _CH and _EA_IN + _L <= _CH


def _zero_fill(o_hbm, zbuf, sem, start, total):
    off = 0
    while off < total:
        n = min(_ZB, total - off)
        pltpu.make_async_copy(
            zbuf.at[pl.ds(0, n)], o_hbm.at[pl.ds(start + off, n)], sem
        ).start()
        off += n


def _sc_body(x_hbm, o_hbm, *rest):
    bufs = rest[:_NBUF]
    zbuf = rest[_NBUF]
    in_sems, out_sems = rest[_NBUF + 1], rest[_NBUF + 2]
    cid = lax.axis_index("c")
    sid = lax.axis_index("s")
    wid = sid * _NC + cid

    def _zb(i, carry):
        z = jnp.zeros((_L,), jnp.float32)
        base = i * (4 * _L)
        zbuf[pl.ds(base, _L)] = z
        zbuf[pl.ds(base + _L, _L)] = z
        zbuf[pl.ds(base + 2 * _L, _L)] = z
        zbuf[pl.ds(base + 3 * _L, _L)] = z
        return carry

    lax.fori_loop(0, _ZB // (4 * _L), _zb, 0)
    lane = lax.broadcasted_iota(jnp.int32, (_L,), 0)

    def chunk_idx(j):
        return wid + _NW * j

    def not_full(j):
        m = chunk_idx(j)
        return jnp.logical_not((m >= _FE_LO) & (m < _FE_HI))

    def in_desc(j):
        b = j % _NBUF
        return pltpu.make_async_copy(
            x_hbm.at[pl.ds(chunk_idx(j) * _CH, _CH)], bufs[b],
            in_sems.at[b])

    def start_in(j):
        @pl.when(not_full(j))
        def _():
            in_desc(j).start()

    def drain_out(b):
        pltpu.make_async_copy(
            x_hbm.at[pl.ds(0, _CH)], bufs[b], out_sems.at[b]
        ).wait()

    for k in range(min(_NBUF - 1, _CPW)):
        start_in(k)

    for j in range(_CPW):
        b = j % _NBUF
        buf = bufs[b]
        osem = out_sems.at[b]
        m = chunk_idx(j)
        lo = m * _CH
        fully_erased = (m >= _FE_LO) & (m < _FE_HI)
        is_b = m == _MB
        is_e = m == _ME
        plain = jnp.logical_not(fully_erased | is_b | is_e)

        @pl.when(jnp.logical_not(fully_erased))
        def _wait_in():
            in_desc(j).wait()

        @pl.when(plain)
        def _store_plain():
            pltpu.make_async_copy(buf, o_hbm.at[pl.ds(lo, _CH)], osem).start()

        @pl.when(fully_erased)
        def _store_zeros():
            _zero_fill(o_hbm, zbuf, osem, lo, _CH)

        @pl.when(is_b)
        def _store_begin():
            base = _B_IN // _L * _L
            v = buf[pl.ds(base, _L)]
            buf[pl.ds(base, _L)] = jnp.where(
                lane >= _B_IN - base, jnp.float32(0.0), v)
            pltpu.make_async_copy(
                buf.at[pl.ds(0, _BA_IN)], o_hbm.at[pl.ds(lo, _BA_IN)], osem
            ).start()
            _zero_fill(o_hbm, zbuf, osem, lo + _BA_IN, _CH - _BA_IN)

        @pl.when(is_e)
        def _store_end():
            v = buf[pl.ds(_EA_IN, _L)]
            buf[pl.ds(_EA_IN, _L)] = jnp.where(
                lane < _E_IN - _EA_IN, jnp.float32(0.0), v)
            _zero_fill(o_hbm, zbuf, osem, lo, _EA_IN)
            pltpu.make_async_copy(
                buf.at[pl.ds(_EA_IN, _CH - _EA_IN)],
                o_hbm.at[pl.ds(lo + _EA_IN, _CH - _EA_IN)], osem,
            ).start()

        nxt = j + _NBUF - 1
        if nxt < _CPW:
            if j >= 1:
                drain_out((j - 1) % _NBUF)
            start_in(nxt)

    for j in range(max(_CPW - _NBUF, 0), _CPW):
        drain_out(j % _NBUF)


_sc_call = functools.partial(
    pl.kernel,
    out_type=jax.ShapeDtypeStruct((_N,), jnp.float32),
    mesh=plsc.VectorSubcoreMesh(core_axis_name="c", subcore_axis_name="s"),
    scratch_types=(
        [pltpu.VMEM((_CH,), jnp.float32) for _ in range(_NBUF)]
        + [
            pltpu.VMEM((_ZB,), jnp.float32),
            pltpu.SemaphoreType.DMA((_NBUF,)),
            pltpu.SemaphoreType.DMA((_NBUF,)),
        ]
    ),
)(_sc_body)


def kernel(vector):
    return _sc_call(vector)
